# baseline (speedup 1.0000x reference)
import functools

import jax
import jax.numpy as jnp
from jax import lax
from jax.experimental import pallas as pl
from jax.experimental.pallas import tpu as pltpu
from jax.experimental.pallas import tpu_sc as plsc

_B, _S, _D, _V = 4, 4096, 1024, 100000
_L = 16
_NC, _NS = 2, 16
_NW = _NC * _NS
_TOK = _B * _S
_TPW = _TOK // _NW
_C = 16
_NCH = _TPW // _C
_VREGS = _D // _L
_SCALE = float(_D) ** 0.5
_NBUF = 3


def _pe_table():
    pos = jnp.arange(_S, dtype=jnp.float32)[:, None]
    i = jnp.arange(0, _D, 2, dtype=jnp.float32)
    angle = pos / jnp.power(10000.0, i / _D)
    pe = jnp.zeros((_S, _D), dtype=jnp.float32)
    pe = pe.at[:, 0::2].set(jnp.sin(angle))
    pe = pe.at[:, 1::2].set(jnp.cos(angle))
    return pe


def _sc_embed(x_flat, table, pe, gamma, beta):
    mesh = plsc.VectorSubcoreMesh(core_axis_name="c", subcore_axis_name="s")

    @functools.partial(
        pl.kernel,
        mesh=mesh,
        out_type=jax.ShapeDtypeStruct((_TOK, _D), jnp.float32),
        scratch_types=[
            pltpu.VMEM((_TPW,), jnp.int32),
            [pltpu.VMEM((_C, _D), jnp.float32) for _ in range(_NBUF)],
            [pltpu.VMEM((_C, _D), jnp.float32) for _ in range(_NBUF)],
            pltpu.VMEM((_D,), jnp.float32),
            pltpu.VMEM((_D,), jnp.float32),
            [pltpu.SemaphoreType.DMA for _ in range(_NBUF)],
            [pltpu.SemaphoreType.DMA for _ in range(_NBUF)],
            [pltpu.SemaphoreType.DMA for _ in range(_NBUF)],
        ],
        compiler_params=pltpu.CompilerParams(needs_layout_passes=False),
    )
    def k(x_hbm, table_hbm, pe_hbm, gamma_hbm, beta_hbm, out_hbm,
          idx_all, rows, pes, g_v, b_v, gsem, psem, osem):
        wid = lax.axis_index("s") * _NC + lax.axis_index("c")
        tok0 = wid * _TPW
        pos0 = tok0 % _S
        pltpu.sync_copy(gamma_hbm, g_v)
        pltpu.sync_copy(beta_hbm, b_v)
        pltpu.sync_copy(x_hbm.at[pl.ds(tok0, _TPW)], idx_all)
        riota = jax.lax.broadcasted_iota(jnp.int32, (_L,), 0)

        def gather_desc(c, bb):
            idx_ref = idx_all.at[pl.ds(c * _C, _C)]
            return pltpu.make_async_copy(table_hbm.at[idx_ref], rows[bb],
                                         gsem[bb])

        def pe_desc(c, bb):
            src = pe_hbm.at[pl.ds(pos0 + c * _C, _C)]
            return pltpu.make_async_copy(src, pes[bb], psem[bb])

        def out_desc(c, bb):
            dst = out_hbm.at[pl.ds(tok0 + c * _C, _C)]
            return pltpu.make_async_copy(rows[bb], dst, osem[bb])

        def issue(c, bb):
            gather_desc(c, bb).start()
            pe_desc(c, bb).start()

        def compute(c, bb):
            gather_desc(c, bb).wait()
            pe_desc(c, bb).wait()
            rbuf, pbuf = rows[bb], pes[bb]

            def p1(j, carry):
                accs = list(carry[:4])
                sqs = list(carry[4:])
                colbase = jnp.full((_L,), j * _L, jnp.int32)
                for t in range(_L):
                    col = colbase + t
                    v = plsc.load_gather(rbuf, [riota, col])
                    p = plsc.load_gather(pbuf, [riota, col])
                    v = v * _SCALE + p
                    plsc.store_scatter(rbuf, [riota, col], v)
                    a = t % 4
                    accs[a] = accs[a] + v
                    sqs[a] = sqs[a] + v * v
                return tuple(accs) + tuple(sqs)

            z = jnp.zeros((_L,), jnp.float32)
            res = lax.fori_loop(0, _VREGS, p1, (z,) * 8)
            s1 = (res[0] + res[1]) + (res[2] + res[3])
            s2 = (res[4] + res[5]) + (res[6] + res[7])
            mean = s1 * (1.0 / _D)
            var = s2 * (1.0 / _D) - mean * mean
            xv = var + 1e-5
            iv = lax.bitcast_convert_type(xv, jnp.int32)
            iv = jnp.int32(0x5F3759DF) - lax.shift_right_logical(iv, 1)
            y = lax.bitcast_convert_type(iv, jnp.float32)
            y = y * (1.5 - 0.5 * xv * y * y)
            y = y * (1.5 - 0.5 * xv * y * y)
            y = y * (1.5 - 0.5 * xv * y * y)
            ms = mean * y

            def p2(j, _):
                gvec = g_v[pl.ds(j * _L, _L)]
                bvec = b_v[pl.ds(j * _L, _L)]
                colbase = jnp.full((_L,), j * _L, jnp.int32)
                for t in range(_L):
                    col = colbase + t
                    v = plsc.load_gather(rbuf, [riota, col])
                    o = (v * y - ms) * gvec[t] + bvec[t]
                    plsc.store_scatter(rbuf, [riota, col], o)
                return 0

            lax.fori_loop(0, _VREGS, p2, 0)

        issue(0, 0)
        issue(1, 1)
        compute(0, 0)
        out_desc(0, 0).start()
        issue(2, 2)

        def loop_body(i, _):
            for b in range(3):
                c = 3 * i + 1 + b
                bb = (1 + b) % 3
                pf = b
                compute(c, bb)
                out_desc(c, bb).start()
                out_desc(c - 1, pf).wait()
                issue(c + 2, pf)
            return 0

        lax.fori_loop(0, 9, loop_body, 0)
        compute(28, 1)
        out_desc(28, 1).start()
        out_desc(27, 0).wait()
        issue(30, 0)
        compute(29, 2)
        out_desc(29, 2).start()
        out_desc(28, 1).wait()
        issue(31, 1)
        compute(30, 0)
        out_desc(30, 0).start()
        compute(31, 1)
        out_desc(31, 1).start()
        out_desc(29, 2).wait()
        out_desc(30, 0).wait()
        out_desc(31, 1).wait()

    return k(x_flat, table, pe, gamma, beta)


def kernel(x, table, gamma, beta):
    pe = _pe_table()
    out = _sc_embed(x.reshape(-1), table, pe, gamma, beta)
    return out.reshape(_B, _S, _D)


# speedup vs baseline: 6.1288x; 6.1288x over previous
import functools

import jax
import jax.numpy as jnp
from jax import lax
from jax.experimental import pallas as pl
from jax.experimental.pallas import tpu as pltpu
from jax.experimental.pallas import tpu_sc as plsc

_B, _S, _D, _V = 4, 4096, 1024, 100000
_L = 16
_NC, _NS = 2, 16
_NW = _NC * _NS
_TOK = _B * _S
_TPW = _TOK // _NW
_C = 16
_NCH = _TPW // _C
_VREGS = _D // _L
_SCALE = float(_D) ** 0.5
_NBUF = 2


def _pe_table():
    pos = jnp.arange(_S, dtype=jnp.float32)[:, None]
    i = jnp.arange(0, _D, 2, dtype=jnp.float32)
    angle = pos / jnp.power(10000.0, i / _D)
    pe = jnp.zeros((_S, _D), dtype=jnp.float32)
    pe = pe.at[:, 0::2].set(jnp.sin(angle))
    pe = pe.at[:, 1::2].set(jnp.cos(angle))
    return pe


def _sc_embed(x_flat, table, pe, gamma, beta):
    mesh = plsc.VectorSubcoreMesh(core_axis_name="c", subcore_axis_name="s")

    @functools.partial(
        pl.kernel,
        mesh=mesh,
        out_type=jax.ShapeDtypeStruct((_TOK, _D), jnp.float32),
        scratch_types=[
            pltpu.VMEM((_TPW,), jnp.int32),
            [pltpu.VMEM((_C, _D), jnp.float32) for _ in range(_NBUF)],
            [pltpu.VMEM((_C, _D), jnp.float32) for _ in range(_NBUF)],
            pltpu.VMEM((_D,), jnp.float32),
            pltpu.VMEM((_D,), jnp.float32),
            [pltpu.SemaphoreType.DMA for _ in range(_NBUF)],
            [pltpu.SemaphoreType.DMA for _ in range(_NBUF)],
        ],
        compiler_params=pltpu.CompilerParams(needs_layout_passes=False),
    )
    def k(x_hbm, table_hbm, pe_hbm, gamma_hbm, beta_hbm, out_hbm,
          idx_all, rows, pes, g_v, b_v, gsem, psem):
        wid = lax.axis_index("s") * _NC + lax.axis_index("c")
        tok0 = wid * _TPW
        pos0 = tok0 % _S
        pltpu.sync_copy(gamma_hbm, g_v)
        pltpu.sync_copy(beta_hbm, b_v)
        pltpu.sync_copy(x_hbm.at[pl.ds(tok0, _TPW)], idx_all)
        riota = jax.lax.broadcasted_iota(jnp.int32, (_L,), 0)

        def gather_desc(c, bb):
            idx_ref = idx_all.at[pl.ds(c * _C, _C)]
            return pltpu.make_async_copy(table_hbm.at[idx_ref], rows[bb],
                                         gsem[bb])

        def pe_desc(c, bb):
            src = pe_hbm.at[pl.ds(pos0 + c * _C, _C)]
            return pltpu.make_async_copy(src, pes[bb], psem[bb])

        def issue(c, bb):
            gather_desc(c, bb).start()
            pe_desc(c, bb).start()

        def compute(c, bb):
            gather_desc(c, bb).wait()
            pe_desc(c, bb).wait()
            rbuf, pbuf = rows[bb], pes[bb]

            rstds = []
            mss = []
            for r in range(_C):

                def p1(jj, carry, r=r):
                    a0, a1, q0, q1 = carry
                    s0 = pl.ds(jj * 2 * _L, _L)
                    s1_ = pl.ds((jj * 2 + 1) * _L, _L)
                    v0 = rbuf[r, s0] * _SCALE + pbuf[r, s0]
                    v1 = rbuf[r, s1_] * _SCALE + pbuf[r, s1_]
                    rbuf[r, s0] = v0
                    rbuf[r, s1_] = v1
                    return a0 + v0, a1 + v1, q0 + v0 * v0, q1 + v1 * v1

                z = jnp.zeros((_L,), jnp.float32)
                a0, a1, q0, q1 = lax.fori_loop(0, _VREGS // 2, p1,
                                               (z, z, z, z), unroll=2)
                s1 = jnp.sum(a0 + a1)
                s2 = jnp.sum(q0 + q1)
                mean = s1 * (1.0 / _D)
                var = s2 * (1.0 / _D) - mean * mean
                xs = var + 1e-5
                ii = lax.bitcast_convert_type(xs, jnp.int32)
                ii = jnp.int32(0x5F3759DF) - lax.shift_right_logical(ii, 1)
                ys = lax.bitcast_convert_type(ii, jnp.float32)
                ys = ys * (1.5 - 0.5 * xs * ys * ys)
                ys = ys * (1.5 - 0.5 * xs * ys * ys)
                ys = ys * (1.5 - 0.5 * xs * ys * ys)
                rstds.append(ys)
                mss.append(mean * ys)

            def p2(j, _):
                sl = pl.ds(j * _L, _L)
                gvec = g_v[sl]
                bvec = b_v[sl]
                for r in range(_C):
                    v = rbuf[r, sl]
                    rbuf[r, sl] = (v * rstds[r] - mss[r]) * gvec + bvec
                return 0

            lax.fori_loop(0, _VREGS, p2, 0)

        issue(0, 0)

        def loop_body(i, _):
            for b in range(2):
                c = 2 * i + b
                nb = 1 - b

                @pl.when(c + 1 < _NCH)
                def _():
                    issue(c + 1, nb)

                compute(c, b)
                pltpu.sync_copy(rows[b], out_hbm.at[pl.ds(tok0 + c * _C, _C)])
            return 0

        lax.fori_loop(0, _NCH // 2, loop_body, 0)

    return k(x_flat, table, pe, gamma, beta)


def kernel(x, table, gamma, beta):
    pe = _pe_table()
    out = _sc_embed(x.reshape(-1), table, pe, gamma, beta)
    return out.reshape(_B, _S, _D)


# speedup vs baseline: 9.4074x; 1.5349x over previous
import functools

import jax
import jax.numpy as jnp
from jax import lax
from jax.experimental import pallas as pl
from jax.experimental.pallas import tpu as pltpu
from jax.experimental.pallas import tpu_sc as plsc

_B, _S, _D, _V = 4, 4096, 1024, 100000
_L = 16
_NC, _NS = 2, 16
_NW = _NC * _NS
_TOK = _B * _S
_TPW = _TOK // _NW
_C = 16
_NCH = _TPW // _C
_VREGS = _D // _L
_SCALE = float(_D) ** 0.5
_NBUF = 3


import numpy as np


def _pe_table():
    pos = np.arange(_S, dtype=np.float32)[:, None]
    i = np.arange(0, _D, 2, dtype=np.float32)
    angle = (pos / np.power(np.float32(10000.0), i / np.float32(_D))).astype(np.float32)
    pe = np.zeros((_S, _D), dtype=np.float32)
    pe[:, 0::2] = np.sin(angle)
    pe[:, 1::2] = np.cos(angle)
    return pe


_PE = _pe_table()


def _sc_embed(x_flat, table, pe, gamma, beta):
    mesh = plsc.VectorSubcoreMesh(core_axis_name="c", subcore_axis_name="s")

    @functools.partial(
        pl.kernel,
        mesh=mesh,
        out_type=jax.ShapeDtypeStruct((_TOK, _D), jnp.float32),
        scratch_types=[
            pltpu.VMEM((_TPW,), jnp.int32),
            [pltpu.VMEM((_C, _D), jnp.float32) for _ in range(_NBUF)],
            [pltpu.VMEM((_C, _D), jnp.float32) for _ in range(_NBUF)],
            pltpu.VMEM((_D,), jnp.float32),
            pltpu.VMEM((_D,), jnp.float32),
            [pltpu.SemaphoreType.DMA for _ in range(_NBUF)],
            [pltpu.SemaphoreType.DMA for _ in range(_NBUF)],
            [pltpu.SemaphoreType.DMA for _ in range(_NBUF)],
        ],
        compiler_params=pltpu.CompilerParams(needs_layout_passes=False),
    )
    def k(x_hbm, table_hbm, pe_hbm, gamma_hbm, beta_hbm, out_hbm,
          idx_all, rows, pes, g_v, b_v, gsem, psem, osem):
        wid = lax.axis_index("s") * _NC + lax.axis_index("c")
        tok0 = wid * _TPW
        pos0 = tok0 % _S
        pltpu.sync_copy(gamma_hbm, g_v)
        pltpu.sync_copy(beta_hbm, b_v)
        pltpu.sync_copy(x_hbm.at[pl.ds(tok0, _TPW)], idx_all)
        riota = jax.lax.broadcasted_iota(jnp.int32, (_L,), 0)

        def gather_desc(c, bb):
            idx_ref = idx_all.at[pl.ds(c * _C, _C)]
            return pltpu.make_async_copy(table_hbm.at[idx_ref], rows[bb],
                                         gsem[bb])

        def pe_desc(c, bb):
            src = pe_hbm.at[pl.ds(pos0 + c * _C, _C)]
            return pltpu.make_async_copy(src, pes[bb], psem[bb])

        def issue(c, bb):
            gather_desc(c, bb).start()
            pe_desc(c, bb).start()

        def out_desc(c, bb):
            dst = out_hbm.at[pl.ds(tok0 + c * _C, _C)]
            return pltpu.make_async_copy(rows[bb], dst, osem[bb])

        def compute(c, bb):
            gather_desc(c, bb).wait()
            pe_desc(c, bb).wait()
            rbuf, pbuf = rows[bb], pes[bb]

            rstds = []
            mss = []
            for r in range(_C):

                def p1(jj, carry, r=r):
                    a0, a1, q0, q1 = carry
                    s0 = pl.ds(jj * 2 * _L, _L)
                    s1_ = pl.ds((jj * 2 + 1) * _L, _L)
                    v0 = rbuf[r, s0] * _SCALE + pbuf[r, s0]
                    v1 = rbuf[r, s1_] * _SCALE + pbuf[r, s1_]
                    rbuf[r, s0] = v0
                    rbuf[r, s1_] = v1
                    return a0 + v0, a1 + v1, q0 + v0 * v0, q1 + v1 * v1

                z = jnp.zeros((_L,), jnp.float32)
                a0, a1, q0, q1 = lax.fori_loop(0, _VREGS // 2, p1,
                                               (z, z, z, z), unroll=2)
                s1 = jnp.sum(a0 + a1)
                s2 = jnp.sum(q0 + q1)
                mean = s1 * (1.0 / _D)
                var = s2 * (1.0 / _D) - mean * mean
                xs = var + 1e-5
                ii = lax.bitcast_convert_type(xs, jnp.int32)
                ii = jnp.int32(0x5F3759DF) - lax.shift_right_logical(ii, 1)
                ys = lax.bitcast_convert_type(ii, jnp.float32)
                ys = ys * (1.5 - 0.5 * xs * ys * ys)
                ys = ys * (1.5 - 0.5 * xs * ys * ys)
                ys = ys * (1.5 - 0.5 * xs * ys * ys)
                rstds.append(ys)
                mss.append(mean * ys)

            def p2(j, _):
                sl = pl.ds(j * _L, _L)
                gvec = g_v[sl]
                bvec = b_v[sl]
                for r in range(_C):
                    v = rbuf[r, sl]
                    rbuf[r, sl] = (v * rstds[r] - mss[r]) * gvec + bvec
                return 0

            lax.fori_loop(0, _VREGS, p2, 0)

        issue(0, 0)
        issue(1, 1)
        compute(0, 0)
        out_desc(0, 0).start()
        issue(2, 2)

        def loop_body(i, _):
            for b in range(3):
                c = 3 * i + 1 + b
                bb = (1 + b) % 3
                pf = b
                compute(c, bb)
                out_desc(c, bb).start()

                @pl.when(c + 2 < _NCH)
                def _():
                    out_desc(c - 1, pf).wait()
                    issue(c + 2, pf)

            return 0

        lax.fori_loop(0, 10, loop_body, 0)
        compute(31, 1)
        out_desc(31, 1).start()
        out_desc(29, 2).wait()
        out_desc(30, 0).wait()
        out_desc(31, 1).wait()

    return k(x_flat, table, pe, gamma, beta)


def kernel(x, table, gamma, beta):
    pe = jnp.asarray(_PE)
    out = _sc_embed(x.reshape(-1), table, pe, gamma, beta)
    return out.reshape(_B, _S, _D)
